# baseline (speedup 1.0000x reference)
import jax
import jax.numpy as jnp
from jax import lax
from jax.experimental import pallas as pl
from jax.experimental.pallas import tpu as pltpu
from jax.experimental.pallas import tpu_sc as plsc

N_NODES = 10000
N_EDGES = 320000
D_FEAT = 128
HIDDEN = 64
BN_EPS = 1e-5
NEG_SLOPE = 0.01

NC = 2
NS = 16
NW = NC * NS
E_HALF = N_EDGES // 2
EDGES_PER_W = E_HALF // NW
GCHUNK = 40
NCHUNK = EDGES_PER_W // GCHUNK
CHUNK = 80

PACK = 2 * HIDDEN
ACC_ROWS = N_NODES // 2 + 8
DUMP_STRIPE = 312
DUMP_LAST = ACC_ROWS - DUMP_STRIPE * (NS - 1)


def _lrelu(h):
    return jnp.maximum(h, NEG_SLOPE * h)


def _gather_body(src2_hbm, dst2_hbm, x_hbm, xs_out, xd_out, *scr):
    ia = scr[0]
    ib = scr[1]
    buf_s = scr[2:5]
    buf_d = scr[5:8]
    gsem_s = scr[8:11]
    gsem_d = scr[11:14]
    wsem_s = scr[14:17]
    wsem_d = scr[17:20]

    wid = lax.axis_index("s") * NC + lax.axis_index("c")
    base = wid * EDGES_PER_W

    pltpu.sync_copy(src2_hbm.at[wid], ia)
    pltpu.sync_copy(dst2_hbm.at[wid], ib)

    def start(c, k):
        pltpu.async_copy(x_hbm.at[ia.at[c]], buf_s[k], gsem_s[k])
        pltpu.async_copy(x_hbm.at[ib.at[c]], buf_d[k], gsem_d[k])

    def wait_gather(c, k):
        pltpu.make_async_copy(x_hbm.at[ia.at[c]], buf_s[k], gsem_s[k]).wait()
        pltpu.make_async_copy(x_hbm.at[ib.at[c]], buf_d[k], gsem_d[k]).wait()

    def start_writeback(c, k):
        off = base + c * GCHUNK
        pltpu.async_copy(buf_s[k], xs_out.at[pl.ds(off, GCHUNK)], wsem_s[k])
        pltpu.async_copy(buf_d[k], xd_out.at[pl.ds(off, GCHUNK)], wsem_d[k])

    def wait_writeback(c, k):
        off = base + c * GCHUNK
        pltpu.make_async_copy(buf_s[k], xs_out.at[pl.ds(off, GCHUNK)],
                              wsem_s[k]).wait()
        pltpu.make_async_copy(buf_d[k], xd_out.at[pl.ds(off, GCHUNK)],
                              wsem_d[k]).wait()

    start(0, 0)
    start(1, 1)

    def body(j, carry):
        c0 = 3 * j
        for k in range(3):
            c = c0 + k
            s2 = (k + 2) % 3
            if k == 0:
                @pl.when(j >= 1)
                def _():
                    wait_writeback(c - 1, s2)
            else:
                wait_writeback(c - 1, s2)
            start(c + 2, s2)
            wait_gather(c, k)
            start_writeback(c, k)
        return carry

    lax.fori_loop(0, (NCHUNK - 2) // 3, body, 0)

    wait_gather(NCHUNK - 2, 0)
    start_writeback(NCHUNK - 2, 0)
    wait_gather(NCHUNK - 1, 1)
    start_writeback(NCHUNK - 1, 1)
    wait_writeback(NCHUNK - 3, 2)
    wait_writeback(NCHUNK - 2, 0)
    wait_writeback(NCHUNK - 1, 1)


def _gather(src2, dst2, x):
    mesh = plsc.VectorSubcoreMesh(core_axis_name="c", subcore_axis_name="s")
    f32 = jnp.float32
    kern = pl.kernel(
        _gather_body,
        mesh=mesh,
        out_type=(
            jax.ShapeDtypeStruct((E_HALF, D_FEAT), f32),
            jax.ShapeDtypeStruct((E_HALF, D_FEAT), f32),
        ),
        scratch_types=(
            [pltpu.VMEM((NCHUNK, GCHUNK), jnp.int32)] * 2
            + [pltpu.VMEM((GCHUNK, D_FEAT), f32)] * 6
            + [pltpu.SemaphoreType.DMA] * 12
        ),
    )
    return kern(src2, dst2, x)


def _mlp_body(xs_ref, xd_ref, w_ref, s1a_ref, t1a_ref, w1a_ref,
              s1b_ref, t1b_ref, w1b_ref, b1_ref,
              s2_ref, t2_ref, w2_ref, b2_ref, m_ref):
    xi = xd_ref[...]
    xj = xs_ref[...]
    ha = jnp.dot(_lrelu(xi * s1a_ref[...] + t1a_ref[...]), w1a_ref[...],
                 preferred_element_type=jnp.float32)
    hb = jnp.dot(_lrelu((xj - xi) * s1b_ref[...] + t1b_ref[...]), w1b_ref[...],
                 preferred_element_type=jnp.float32)
    h1 = ha + hb + b1_ref[...]
    h2 = jnp.dot(_lrelu(h1 * s2_ref[...] + t2_ref[...]), w2_ref[...],
                 preferred_element_type=jnp.float32) + b2_ref[...]
    m_ref[:, :HIDDEN] = h2 * w_ref[:, 0:1]
    m_ref[:, HIDDEN:] = h2 * w_ref[:, 1:2]


def _mlp(xs, xd, w2c, s1a, t1a, w1a, s1b, t1b, w1b, b1, s2, t2, w2, b2):
    rows = 2000
    grid = E_HALF // rows
    vec = lambda n: pl.BlockSpec((1, n), lambda i: (0, 0))
    return pl.pallas_call(
        _mlp_body,
        grid=(grid,),
        in_specs=[
            pl.BlockSpec((rows, D_FEAT), lambda i: (i, 0)),
            pl.BlockSpec((rows, D_FEAT), lambda i: (i, 0)),
            pl.BlockSpec((rows, 2), lambda i: (i, 0)),
            vec(D_FEAT), vec(D_FEAT),
            pl.BlockSpec((D_FEAT, HIDDEN), lambda i: (0, 0)),
            vec(D_FEAT), vec(D_FEAT),
            pl.BlockSpec((D_FEAT, HIDDEN), lambda i: (0, 0)),
            vec(HIDDEN), vec(HIDDEN), vec(HIDDEN),
            pl.BlockSpec((HIDDEN, HIDDEN), lambda i: (0, 0)),
            vec(HIDDEN),
        ],
        out_specs=pl.BlockSpec((rows, PACK), lambda i: (i, 0)),
        out_shape=jax.ShapeDtypeStruct((E_HALF, PACK), jnp.float32),
    )(xs, xd, w2c, s1a, t1a, w1a, s1b, t1b, w1b, b1, s2, t2, w2, b2)


EDGES_PER_SC_TILE = N_EDGES // NC // NS
NCHUNK_SC = EDGES_PER_SC_TILE // CHUNK


def _scatter_body(dst2_hbm, ma_hbm, mb_hbm, out_hbm, *scr):
    idx2 = scr[0]
    m_buf = scr[1:4]
    lsem_m = scr[4:7]
    ssem = scr[7:10]
    z_buf = scr[10]
    acc = scr[11]

    cid = lax.axis_index("c")
    sid = lax.axis_index("s")
    base = sid * EDGES_PER_SC_TILE
    widsc = cid * NS + sid

    pltpu.sync_copy(dst2_hbm.at[widsc], idx2)

    def tr(i, carry):
        r = i // (CHUNK // 16)
        g = (i % (CHUNK // 16)) * 16
        idx2[r, pl.ds(g, 16)] = \
            lax.shift_right_logical(idx2[r, pl.ds(g, 16)], 1)
        return carry

    lax.fori_loop(0, NCHUNK_SC * (CHUNK // 16), tr, 0)

    def zero(i, carry):
        r = i // (PACK // 16)
        c = (i % (PACK // 16)) * 16
        z_buf[r, pl.ds(c, 16)] = jnp.zeros((16,), jnp.float32)
        return carry

    lax.fori_loop(0, DUMP_LAST * (PACK // 16), zero, 0)

    @pl.when(sid < NS - 1)
    def _():
        pltpu.sync_copy(z_buf.at[pl.ds(0, DUMP_STRIPE)],
                        acc.at[pl.ds(sid * DUMP_STRIPE, DUMP_STRIPE)])

    @pl.when(sid == NS - 1)
    def _():
        pltpu.sync_copy(z_buf, acc.at[pl.ds((NS - 1) * DUMP_STRIPE, DUMP_LAST)])

    plsc.subcore_barrier()

    def run_half(m_hbm):
        def start_load(c, k):
            off = base + c * CHUNK
            pltpu.async_copy(m_hbm.at[pl.ds(off, CHUNK)], m_buf[k], lsem_m[k])

        def complete(c, k):
            off = base + c * CHUNK
            pltpu.make_async_copy(m_hbm.at[pl.ds(off, CHUNK)], m_buf[k],
                                  lsem_m[k]).wait()
            pltpu.async_copy(m_buf[k], acc.at[idx2.at[c]], ssem[k], add=True)

        def wait_scatter(c, k):
            pltpu.make_async_copy(m_buf[k], acc.at[idx2.at[c]], ssem[k]).wait()

        start_load(0, 0)
        start_load(1, 1)

        def body(j, carry):
            c0 = 3 * j
            for k in range(3):
                c = c0 + k
                s2 = (k + 2) % 3
                if k == 0:
                    @pl.when(j >= 1)
                    def _():
                        wait_scatter(c - 1, s2)
                else:
                    wait_scatter(c - 1, s2)
                start_load(c + 2, s2)
                complete(c, k)
            return carry

        lax.fori_loop(0, (NCHUNK_SC - 2) // 3, body, 0)

        complete(NCHUNK_SC - 2, 0)
        complete(NCHUNK_SC - 1, 1)
        wait_scatter(NCHUNK_SC - 3, 2)
        wait_scatter(NCHUNK_SC - 2, 0)
        wait_scatter(NCHUNK_SC - 1, 1)

    @pl.when(cid == 0)
    def _():
        run_half(ma_hbm)

    @pl.when(cid == 1)
    def _():
        run_half(mb_hbm)

    plsc.subcore_barrier()

    @pl.when(sid < NS - 1)
    def _():
        pltpu.sync_copy(acc.at[pl.ds(sid * DUMP_STRIPE, DUMP_STRIPE)],
                        out_hbm.at[cid, pl.ds(sid * DUMP_STRIPE, DUMP_STRIPE)])

    @pl.when(sid == NS - 1)
    def _():
        pltpu.sync_copy(acc.at[pl.ds((NS - 1) * DUMP_STRIPE, DUMP_LAST)],
                        out_hbm.at[cid, pl.ds((NS - 1) * DUMP_STRIPE, DUMP_LAST)])


def _scatter(dst2, ma, mb):
    mesh = plsc.VectorSubcoreMesh(core_axis_name="c", subcore_axis_name="s")
    kern = pl.kernel(
        _scatter_body,
        mesh=mesh,
        out_type=jax.ShapeDtypeStruct((NC, ACC_ROWS, PACK), jnp.float32),
        scratch_types=(
            [pltpu.VMEM((NCHUNK_SC, CHUNK), jnp.int32)]
            + [pltpu.VMEM((CHUNK, PACK), jnp.float32)] * 3
            + [pltpu.SemaphoreType.DMA] * 6
            + [pltpu.VMEM((DUMP_LAST, PACK), jnp.float32),
               pltpu.VMEM_SHARED((ACC_ROWS, PACK), jnp.float32)]
        ),
    )
    return kern(dst2, ma, mb)


def _combine_body(p_ref, o_ref):
    o_ref[...] = p_ref[0] + p_ref[1]


def _combine(partials):
    rows = 2504
    grid = ACC_ROWS // rows
    return pl.pallas_call(
        _combine_body,
        grid=(grid,),
        in_specs=[pl.BlockSpec((NC, rows, PACK), lambda i: (0, i, 0))],
        out_specs=pl.BlockSpec((rows, PACK), lambda i: (i, 0)),
        out_shape=jax.ShapeDtypeStruct((ACC_ROWS, PACK), jnp.float32),
    )(partials)


def kernel(x, edge_index, edge_weight, bn1_gamma, bn1_beta, W1, b1,
           bn2_gamma, bn2_beta, W2, b2):
    src = edge_index[0]
    dst = edge_index[1]
    inv = 1.0 / jnp.sqrt(1.0 + BN_EPS)
    s1 = bn1_gamma * inv
    s1a = s1[:D_FEAT].reshape(1, D_FEAT)
    s1b = s1[D_FEAT:].reshape(1, D_FEAT)
    t1a = bn1_beta[:D_FEAT].reshape(1, D_FEAT)
    t1b = bn1_beta[D_FEAT:].reshape(1, D_FEAT)
    s2 = (bn2_gamma * inv).reshape(1, HIDDEN)
    t2 = bn2_beta.reshape(1, HIDDEN)

    parity = (dst & 1).astype(jnp.float32)
    w2c = jnp.stack([edge_weight * (1.0 - parity), edge_weight * parity],
                    axis=1)
    dst2sc = dst.reshape(NW, NCHUNK_SC, CHUNK)

    halves = []
    for h in range(2):
        lo = h * E_HALF
        src3 = lax.slice_in_dim(src, lo, lo + E_HALF).reshape(NW, NCHUNK, GCHUNK)
        dst3 = lax.slice_in_dim(dst, lo, lo + E_HALF).reshape(NW, NCHUNK, GCHUNK)
        xs, xd = _gather(src3, dst3, x)
        w2ch = lax.slice_in_dim(w2c, lo, lo + E_HALF)
        halves.append(_mlp(xs, xd, w2ch,
                           s1a, t1a, W1[:D_FEAT], s1b, t1b, W1[D_FEAT:],
                           b1.reshape(1, HIDDEN), s2, t2, W2,
                           b2.reshape(1, HIDDEN)))
    packed = _combine(_scatter(dst2sc, halves[0], halves[1]))
    return packed.reshape(ACC_ROWS * 2, HIDDEN)[:N_NODES]


# speedup vs baseline: 1.0181x; 1.0181x over previous
import jax
import jax.numpy as jnp
from jax import lax
from jax.experimental import pallas as pl
from jax.experimental.pallas import tpu as pltpu
from jax.experimental.pallas import tpu_sc as plsc

N_NODES = 10000
N_EDGES = 320000
D_FEAT = 128
HIDDEN = 64
BN_EPS = 1e-5
NEG_SLOPE = 0.01

NC = 2
NS = 16
NW = NC * NS
EDGES_PER_W = N_EDGES // NW
CHUNK = 80
NCHUNK = EDGES_PER_W // CHUNK

PACK = 2 * HIDDEN
ACC_ROWS = N_NODES // 2 + 8
DUMP_STRIPE = 312
DUMP_LAST = ACC_ROWS - DUMP_STRIPE * (NS - 1)


def _lrelu(h):
    return jnp.maximum(h, NEG_SLOPE * h)


def _gather_body(src2_hbm, dst2_hbm, x_hbm, xs_out, xd_out, *scr):
    ia = scr[0]
    ib = scr[1]
    buf_s = scr[2:5]
    buf_d = scr[5:8]
    gsem_s = scr[8:11]
    gsem_d = scr[11:14]
    wsem_s = scr[14:17]
    wsem_d = scr[17:20]

    wid = lax.axis_index("s") * NC + lax.axis_index("c")
    base = wid * EDGES_PER_W

    pltpu.sync_copy(src2_hbm.at[wid], ia)
    pltpu.sync_copy(dst2_hbm.at[wid], ib)

    def start(c, k):
        pltpu.async_copy(x_hbm.at[ia.at[c]], buf_s[k], gsem_s[k])
        pltpu.async_copy(x_hbm.at[ib.at[c]], buf_d[k], gsem_d[k])

    def wait_gather(c, k):
        pltpu.make_async_copy(x_hbm.at[ia.at[c]], buf_s[k], gsem_s[k]).wait()
        pltpu.make_async_copy(x_hbm.at[ib.at[c]], buf_d[k], gsem_d[k]).wait()

    def start_writeback(c, k):
        off = base + c * CHUNK
        pltpu.async_copy(buf_s[k], xs_out.at[pl.ds(off, CHUNK)], wsem_s[k])
        pltpu.async_copy(buf_d[k], xd_out.at[pl.ds(off, CHUNK)], wsem_d[k])

    def wait_writeback(c, k):
        off = base + c * CHUNK
        pltpu.make_async_copy(buf_s[k], xs_out.at[pl.ds(off, CHUNK)],
                              wsem_s[k]).wait()
        pltpu.make_async_copy(buf_d[k], xd_out.at[pl.ds(off, CHUNK)],
                              wsem_d[k]).wait()

    start(0, 0)
    start(1, 1)

    def body(j, carry):
        c0 = 3 * j
        for k in range(3):
            c = c0 + k
            s2 = (k + 2) % 3
            if k == 0:
                @pl.when(j >= 1)
                def _():
                    wait_writeback(c - 1, s2)
            else:
                wait_writeback(c - 1, s2)
            start(c + 2, s2)
            wait_gather(c, k)
            start_writeback(c, k)
        return carry

    lax.fori_loop(0, (NCHUNK - 2) // 3, body, 0)

    wait_gather(NCHUNK - 2, 0)
    start_writeback(NCHUNK - 2, 0)
    wait_gather(NCHUNK - 1, 1)
    start_writeback(NCHUNK - 1, 1)
    wait_writeback(NCHUNK - 3, 2)
    wait_writeback(NCHUNK - 2, 0)
    wait_writeback(NCHUNK - 1, 1)


def _gather(src2, dst2, x):
    mesh = plsc.VectorSubcoreMesh(core_axis_name="c", subcore_axis_name="s")
    f32 = jnp.float32
    kern = pl.kernel(
        _gather_body,
        mesh=mesh,
        out_type=(
            jax.ShapeDtypeStruct((N_EDGES, D_FEAT), f32),
            jax.ShapeDtypeStruct((N_EDGES, D_FEAT), f32),
        ),
        scratch_types=(
            [pltpu.VMEM((NCHUNK, CHUNK), jnp.int32)] * 2
            + [pltpu.VMEM((CHUNK, D_FEAT), f32)] * 6
            + [pltpu.SemaphoreType.DMA] * 12
        ),
    )
    return kern(src2, dst2, x)


def _mlp_body(xs_ref, xd_ref, w_ref, s1a_ref, t1a_ref, w1a_ref,
              s1b_ref, t1b_ref, w1b_ref, b1_ref,
              s2_ref, t2_ref, w2_ref, b2_ref, m_ref):
    xi = xd_ref[...]
    xj = xs_ref[...]
    ha = jnp.dot(_lrelu(xi * s1a_ref[...] + t1a_ref[...]), w1a_ref[...],
                 preferred_element_type=jnp.float32)
    hb = jnp.dot(_lrelu((xj - xi) * s1b_ref[...] + t1b_ref[...]), w1b_ref[...],
                 preferred_element_type=jnp.float32)
    h1 = ha + hb + b1_ref[...]
    h2 = jnp.dot(_lrelu(h1 * s2_ref[...] + t2_ref[...]), w2_ref[...],
                 preferred_element_type=jnp.float32) + b2_ref[...]
    m_ref[:, :HIDDEN] = h2 * w_ref[:, 0:1]
    m_ref[:, HIDDEN:] = h2 * w_ref[:, 1:2]


def _mlp(xs, xd, w2c, s1a, t1a, w1a, s1b, t1b, w1b, b1, s2, t2, w2, b2):
    rows = 2560
    grid = N_EDGES // rows
    vec = lambda n: pl.BlockSpec((1, n), lambda i: (0, 0))
    return pl.pallas_call(
        _mlp_body,
        grid=(grid,),
        in_specs=[
            pl.BlockSpec((rows, D_FEAT), lambda i: (i, 0)),
            pl.BlockSpec((rows, D_FEAT), lambda i: (i, 0)),
            pl.BlockSpec((rows, 2), lambda i: (i, 0)),
            vec(D_FEAT), vec(D_FEAT),
            pl.BlockSpec((D_FEAT, HIDDEN), lambda i: (0, 0)),
            vec(D_FEAT), vec(D_FEAT),
            pl.BlockSpec((D_FEAT, HIDDEN), lambda i: (0, 0)),
            vec(HIDDEN), vec(HIDDEN), vec(HIDDEN),
            pl.BlockSpec((HIDDEN, HIDDEN), lambda i: (0, 0)),
            vec(HIDDEN),
        ],
        out_specs=pl.BlockSpec((rows, PACK), lambda i: (i, 0)),
        out_shape=jax.ShapeDtypeStruct((N_EDGES, PACK), jnp.float32),
    )(xs, xd, w2c, s1a, t1a, w1a, s1b, t1b, w1b, b1, s2, t2, w2, b2)


EDGES_PER_SC_TILE = N_EDGES // NC // NS
NCHUNK_SC = EDGES_PER_SC_TILE // CHUNK


def _scatter_body(dst2_hbm, m_hbm, out_hbm, *scr):
    idx2 = scr[0]
    m_buf = scr[1:4]
    lsem_m = scr[4:7]
    ssem = scr[7:10]
    z_buf = scr[10]
    acc = scr[11]

    cid = lax.axis_index("c")
    sid = lax.axis_index("s")
    base = cid * (N_EDGES // NC) + sid * EDGES_PER_SC_TILE
    widsc = cid * NS + sid

    pltpu.sync_copy(dst2_hbm.at[widsc], idx2)

    def tr(i, carry):
        r = i // (CHUNK // 16)
        g = (i % (CHUNK // 16)) * 16
        idx2[r, pl.ds(g, 16)] = \
            lax.shift_right_logical(idx2[r, pl.ds(g, 16)], 1)
        return carry

    lax.fori_loop(0, NCHUNK_SC * (CHUNK // 16), tr, 0)

    def zero(i, carry):
        r = i // (PACK // 16)
        c = (i % (PACK // 16)) * 16
        z_buf[r, pl.ds(c, 16)] = jnp.zeros((16,), jnp.float32)
        return carry

    lax.fori_loop(0, DUMP_LAST * (PACK // 16), zero, 0)

    @pl.when(sid < NS - 1)
    def _():
        pltpu.sync_copy(z_buf.at[pl.ds(0, DUMP_STRIPE)],
                        acc.at[pl.ds(sid * DUMP_STRIPE, DUMP_STRIPE)])

    @pl.when(sid == NS - 1)
    def _():
        pltpu.sync_copy(z_buf, acc.at[pl.ds((NS - 1) * DUMP_STRIPE, DUMP_LAST)])

    plsc.subcore_barrier()

    def start_load(c, k):
        off = base + c * CHUNK
        pltpu.async_copy(m_hbm.at[pl.ds(off, CHUNK)], m_buf[k], lsem_m[k])

    def complete(c, k):
        off = base + c * CHUNK
        pltpu.make_async_copy(m_hbm.at[pl.ds(off, CHUNK)], m_buf[k],
                              lsem_m[k]).wait()
        pltpu.async_copy(m_buf[k], acc.at[idx2.at[c]], ssem[k], add=True)

    def wait_scatter(c, k):
        pltpu.make_async_copy(m_buf[k], acc.at[idx2.at[c]], ssem[k]).wait()

    start_load(0, 0)
    start_load(1, 1)

    def body(j, carry):
        c0 = 3 * j
        for k in range(3):
            c = c0 + k
            s2 = (k + 2) % 3
            if k == 0:
                @pl.when(j >= 1)
                def _():
                    wait_scatter(c - 1, s2)
            else:
                wait_scatter(c - 1, s2)
            start_load(c + 2, s2)
            complete(c, k)
        return carry

    lax.fori_loop(0, (NCHUNK_SC - 2) // 3, body, 0)

    complete(NCHUNK_SC - 2, 0)
    complete(NCHUNK_SC - 1, 1)
    wait_scatter(NCHUNK_SC - 3, 2)
    wait_scatter(NCHUNK_SC - 2, 0)
    wait_scatter(NCHUNK_SC - 1, 1)
    plsc.subcore_barrier()

    @pl.when(sid < NS - 1)
    def _():
        pltpu.sync_copy(acc.at[pl.ds(sid * DUMP_STRIPE, DUMP_STRIPE)],
                        out_hbm.at[cid, pl.ds(sid * DUMP_STRIPE, DUMP_STRIPE)])

    @pl.when(sid == NS - 1)
    def _():
        pltpu.sync_copy(acc.at[pl.ds((NS - 1) * DUMP_STRIPE, DUMP_LAST)],
                        out_hbm.at[cid, pl.ds((NS - 1) * DUMP_STRIPE, DUMP_LAST)])


def _scatter(dst2, m):
    mesh = plsc.VectorSubcoreMesh(core_axis_name="c", subcore_axis_name="s")
    kern = pl.kernel(
        _scatter_body,
        mesh=mesh,
        out_type=jax.ShapeDtypeStruct((NC, ACC_ROWS, PACK), jnp.float32),
        scratch_types=(
            [pltpu.VMEM((NCHUNK_SC, CHUNK), jnp.int32)]
            + [pltpu.VMEM((CHUNK, PACK), jnp.float32)] * 3
            + [pltpu.SemaphoreType.DMA] * 6
            + [pltpu.VMEM((DUMP_LAST, PACK), jnp.float32),
               pltpu.VMEM_SHARED((ACC_ROWS, PACK), jnp.float32)]
        ),
    )
    return kern(dst2, m)


def _combine_body(p_ref, o_ref):
    o_ref[...] = p_ref[0] + p_ref[1]


def _combine(partials):
    rows = 2504
    grid = ACC_ROWS // rows
    return pl.pallas_call(
        _combine_body,
        grid=(grid,),
        in_specs=[pl.BlockSpec((NC, rows, PACK), lambda i: (0, i, 0))],
        out_specs=pl.BlockSpec((rows, PACK), lambda i: (i, 0)),
        out_shape=jax.ShapeDtypeStruct((ACC_ROWS, PACK), jnp.float32),
    )(partials)


def kernel(x, edge_index, edge_weight, bn1_gamma, bn1_beta, W1, b1,
           bn2_gamma, bn2_beta, W2, b2):
    src = edge_index[0]
    dst = edge_index[1]
    inv = 1.0 / jnp.sqrt(1.0 + BN_EPS)
    s1 = bn1_gamma * inv
    s1a = s1[:D_FEAT].reshape(1, D_FEAT)
    s1b = s1[D_FEAT:].reshape(1, D_FEAT)
    t1a = bn1_beta[:D_FEAT].reshape(1, D_FEAT)
    t1b = bn1_beta[D_FEAT:].reshape(1, D_FEAT)
    s2 = (bn2_gamma * inv).reshape(1, HIDDEN)
    t2 = bn2_beta.reshape(1, HIDDEN)

    src2 = src.reshape(NW, NCHUNK, CHUNK)
    dst2 = dst.reshape(NW, NCHUNK, CHUNK)
    parity = (dst & 1).astype(jnp.float32)
    w2c = jnp.stack([edge_weight * (1.0 - parity), edge_weight * parity],
                    axis=1)

    xs, xd = _gather(src2, dst2, x)
    m = _mlp(xs, xd, w2c,
             s1a, t1a, W1[:D_FEAT], s1b, t1b, W1[D_FEAT:],
             b1.reshape(1, HIDDEN), s2, t2, W2, b2.reshape(1, HIDDEN))
    packed = _combine(_scatter(dst2, m))
    return packed.reshape(ACC_ROWS * 2, HIDDEN)[:N_NODES]
